# baseline (speedup 1.0000x reference)
import functools

import jax
import jax.numpy as jnp
from jax import lax
from jax.experimental import pallas as pl
from jax.experimental.pallas import tpu as pltpu
from jax.experimental.pallas import tpu_sc as plsc

_B, _L, _D = 16384, 200, 32
_GC = 16
_LP = 208


@functools.lru_cache(maxsize=None)
def _build(B, L, D):
    info = plsc.get_sparse_core_info()
    NC, NS = info.num_cores, info.num_subcores
    NW = NC * NS
    BPW = B // NW
    CB = 256
    NCB = BPW // CB

    mesh = plsc.VectorSubcoreMesh(core_axis_name="c", subcore_axis_name="s")

    @functools.partial(
        pl.kernel,
        out_type=jax.ShapeDtypeStruct((B, D), jnp.float32),
        mesh=mesh,
        scratch_types=[
            pltpu.VMEM((CB, _LP), jnp.int32),
            pltpu.VMEM((CB, 16), jnp.int32),
            pltpu.VMEM((4, _LP, D), jnp.float32),
            pltpu.VMEM((CB, D), jnp.float32),
            pltpu.SemaphoreType.DMA,
            pltpu.SemaphoreType.DMA,
            pltpu.SemaphoreType.DMA,
            pltpu.SemaphoreType.DMA,
        ],
        compiler_params=pltpu.CompilerParams(use_tc_tiling_on_sc=False),
    )
    def k(x_hbm, lens_hbm, table_hbm, out_hbm, xv, lens_vm, rows, outb,
          sem0, sem1, sem2, sem3):
        wid = lax.axis_index("s") * NC + lax.axis_index("c")
        base = wid * BPW
        izero = jnp.zeros((16,), jnp.int32)
        zero = jnp.zeros((16,), jnp.float32)
        sems = (sem0, sem1, sem2, sem3)

        def zpad(b, _):
            xv[b, pl.ds(192, 16)] = izero
            return 0
        lax.fori_loop(0, CB, zpad, 0)

        def fire(b, par):
            ln = lens_vm[b, pl.ds(0, 16)][0]
            nch = (ln + (_GC - 1)) // _GC

            def fbody(c, _):
                pltpu.async_copy(
                    table_hbm.at[xv.at[b, pl.ds(c * _GC, _GC)]],
                    rows.at[par, pl.ds(c * _GC, _GC), :],
                    sems[par],
                )
                return 0
            lax.fori_loop(0, nch, fbody, 0)

        def consume(b, par):
            ln = lens_vm[b, pl.ds(0, 16)][0]
            nch = (ln + (_GC - 1)) // _GC

            def drain(c, _):
                pltpu.make_async_copy(
                    table_hbm.at[pl.ds(0, _GC), :],
                    rows.at[par, pl.ds(0, _GC), :],
                    sems[par],
                ).wait()
                return 0
            lax.fori_loop(0, nch, drain, 0)

            nfull = ln // 16
            rem = ln - nfull * 16

            def acc_full(g, accs):
                a0, a1, b0, b1 = accs
                r0 = g * 16
                for j in range(0, 16, 2):
                    a0 = a0 + rows[par, r0 + j, pl.ds(0, 16)]
                    a1 = a1 + rows[par, r0 + j, pl.ds(16, 16)]
                    b0 = b0 + rows[par, r0 + j + 1, pl.ds(0, 16)]
                    b1 = b1 + rows[par, r0 + j + 1, pl.ds(16, 16)]
                return (a0, a1, b0, b1)

            a0, a1, b0, b1 = lax.fori_loop(0, nfull, acc_full,
                                           (zero, zero, zero, zero))
            a0 = a0 + b0
            a1 = a1 + b1

            def acc_rem(j, accs):
                a0, a1 = accs
                r = nfull * 16 + j
                return (a0 + rows[par, r, pl.ds(0, 16)],
                        a1 + rows[par, r, pl.ds(16, 16)])
            a0, a1 = lax.fori_loop(0, rem, acc_rem, (a0, a1))

            vln = jnp.broadcast_to(ln.astype(jnp.float32), (16,))
            outb[b, pl.ds(0, 16)] = a0 / vln
            outb[b, pl.ds(16, 16)] = a1 / vln

        def do_chunk(cb, _):
            gb0 = base + cb * CB
            pltpu.sync_copy(x_hbm.at[pl.ds(gb0, CB), :], xv.at[:, pl.ds(0, L)])
            pltpu.sync_copy(lens_hbm.at[pl.ds(gb0, CB), :], lens_vm)

            fire(jnp.int32(0), 0)
            fire(jnp.int32(1), 1)

            def do_quad(q, _):
                i0 = q * 4
                for j in range(4):
                    b = i0 + j
                    b2 = b + 2
                    pl.when(b2 < CB)(lambda: fire(b2, (j + 2) % 4))
                    consume(b, j)
                return 0

            lax.fori_loop(0, CB // 4, do_quad, 0)

            pltpu.sync_copy(outb, out_hbm.at[pl.ds(gb0, CB), :])
            return 0

        lax.fori_loop(0, NCB, do_chunk, 0)

    return k


def kernel(x, x_lens, embed_table):
    B, L = x.shape
    V, D = embed_table.shape
    k = _build(B, L, D)
    lens_splat = jnp.broadcast_to(x_lens.astype(jnp.int32)[:, None], (B, 16))
    t4 = lax.optimization_barrier(jnp.reshape(embed_table, (V * D // 128, 128)))
    return k(x.astype(jnp.int32), lens_splat, jnp.reshape(t4, (V, D)))


# speedup vs baseline: 1.0370x; 1.0370x over previous
import functools

import jax
import jax.numpy as jnp
from jax import lax
from jax.experimental import pallas as pl
from jax.experimental.pallas import tpu as pltpu
from jax.experimental.pallas import tpu_sc as plsc

_B, _L, _D = 16384, 200, 32
_GC = 16
_LP = 208


@functools.lru_cache(maxsize=None)
def _build(B, L, D):
    info = plsc.get_sparse_core_info()
    NC, NS = info.num_cores, info.num_subcores
    NW = NC * NS
    BPW = B // NW
    CB = 256
    NCB = BPW // CB

    mesh = plsc.VectorSubcoreMesh(core_axis_name="c", subcore_axis_name="s")

    @functools.partial(
        pl.kernel,
        out_type=jax.ShapeDtypeStruct((B, D), jnp.float32),
        mesh=mesh,
        scratch_types=[
            pltpu.VMEM((CB, _LP), jnp.int32),
            pltpu.VMEM((CB, 16), jnp.int32),
            pltpu.VMEM((4, _LP, D), jnp.float32),
            pltpu.VMEM((CB, D), jnp.float32),
            pltpu.SemaphoreType.DMA,
            pltpu.SemaphoreType.DMA,
            pltpu.SemaphoreType.DMA,
            pltpu.SemaphoreType.DMA,
        ],
        compiler_params=pltpu.CompilerParams(use_tc_tiling_on_sc=False),
    )
    def k(x_hbm, lens_hbm, table_hbm, out_hbm, xv, lens_vm, rows, outb,
          sem0, sem1, sem2, sem3):
        wid = lax.axis_index("s") * NC + lax.axis_index("c")
        base = wid * BPW
        izero = jnp.zeros((16,), jnp.int32)
        zero = jnp.zeros((16,), jnp.float32)
        sems = (sem0, sem1, sem2, sem3)

        def zpad(b, _):
            xv[b, pl.ds(192, 16)] = izero
            return 0
        lax.fori_loop(0, CB, zpad, 0)

        def fire(b, par):
            ln = lens_vm[b, pl.ds(0, 16)][0]
            nch = (ln + (_GC - 1)) // _GC

            def fbody(c, _):
                pltpu.async_copy(
                    table_hbm.at[xv.at[b, pl.ds(c * _GC, _GC)]],
                    rows.at[par, pl.ds(c * _GC, _GC), :],
                    sems[par],
                )
                return 0
            lax.fori_loop(0, nch, fbody, 0)

        def consume(b, par):
            ln = lens_vm[b, pl.ds(0, 16)][0]
            nch = (ln + (_GC - 1)) // _GC

            def drain(c, _):
                pltpu.make_async_copy(
                    table_hbm.at[pl.ds(0, _GC), :],
                    rows.at[par, pl.ds(0, _GC), :],
                    sems[par],
                ).wait()
                return 0
            lax.fori_loop(0, nch, drain, 0)

            nfull = ln // 16
            rem = ln - nfull * 16

            def acc_full(g, accs):
                a0, a1, b0, b1 = accs
                r0 = g * 16
                for j in range(0, 16, 2):
                    a0 = a0 + rows[par, r0 + j, pl.ds(0, 16)]
                    a1 = a1 + rows[par, r0 + j, pl.ds(16, 16)]
                    b0 = b0 + rows[par, r0 + j + 1, pl.ds(0, 16)]
                    b1 = b1 + rows[par, r0 + j + 1, pl.ds(16, 16)]
                return (a0, a1, b0, b1)

            a0, a1, b0, b1 = lax.fori_loop(0, nfull, acc_full,
                                           (zero, zero, zero, zero))
            a0 = a0 + b0
            a1 = a1 + b1

            def acc_rem(j, accs):
                a0, a1 = accs
                r = nfull * 16 + j
                return (a0 + rows[par, r, pl.ds(0, 16)],
                        a1 + rows[par, r, pl.ds(16, 16)])
            a0, a1 = lax.fori_loop(0, rem, acc_rem, (a0, a1))

            vln = jnp.broadcast_to(ln.astype(jnp.float32), (16,))
            outb[b, pl.ds(0, 16)] = a0 / vln
            outb[b, pl.ds(16, 16)] = a1 / vln

        def do_chunk(cb, _):
            gb0 = base + cb * CB
            pltpu.sync_copy(x_hbm.at[pl.ds(gb0, CB), :], xv.at[:, pl.ds(0, L)])
            pltpu.sync_copy(lens_hbm.at[pl.ds(gb0, CB), :], lens_vm)

            fire(jnp.int32(0), 0)
            fire(jnp.int32(1), 1)
            fire(jnp.int32(2), 2)

            def do_quad(q, _):
                i0 = q * 4
                for j in range(4):
                    b = i0 + j
                    b3 = b + 3
                    pl.when(b3 < CB)(lambda: fire(b3, (j + 3) % 4))
                    consume(b, j)
                return 0

            lax.fori_loop(0, CB // 4, do_quad, 0)

            pltpu.sync_copy(outb, out_hbm.at[pl.ds(gb0, CB), :])
            return 0

        lax.fori_loop(0, NCB, do_chunk, 0)

    return k


def kernel(x, x_lens, embed_table):
    B, L = x.shape
    V, D = embed_table.shape
    k = _build(B, L, D)
    lens_splat = jnp.broadcast_to(x_lens.astype(jnp.int32)[:, None], (B, 16))
    t4 = lax.optimization_barrier(jnp.reshape(embed_table, (V * D // 128, 128)))
    return k(x.astype(jnp.int32), lens_splat, jnp.reshape(t4, (V, D)))


# speedup vs baseline: 1.0497x; 1.0123x over previous
import functools

import jax
import jax.numpy as jnp
from jax import lax
from jax.experimental import pallas as pl
from jax.experimental.pallas import tpu as pltpu
from jax.experimental.pallas import tpu_sc as plsc

_B, _L, _D = 16384, 200, 32
_GC = 16
_LP = 208


@functools.lru_cache(maxsize=None)
def _build(B, L, D):
    info = plsc.get_sparse_core_info()
    NC, NS = info.num_cores, info.num_subcores
    NW = NC * NS
    BPW = B // NW
    CB = 256
    NCB = BPW // CB

    mesh = plsc.VectorSubcoreMesh(core_axis_name="c", subcore_axis_name="s")

    @functools.partial(
        pl.kernel,
        out_type=jax.ShapeDtypeStruct((B, D), jnp.float32),
        mesh=mesh,
        scratch_types=[
            pltpu.VMEM((CB, _LP), jnp.int32),
            pltpu.VMEM((CB, 16), jnp.int32),
            pltpu.VMEM((8, _LP, D), jnp.float32),
            pltpu.VMEM((CB, D), jnp.float32),
            pltpu.SemaphoreType.DMA,
            pltpu.SemaphoreType.DMA,
            pltpu.SemaphoreType.DMA,
            pltpu.SemaphoreType.DMA,
            pltpu.SemaphoreType.DMA,
            pltpu.SemaphoreType.DMA,
            pltpu.SemaphoreType.DMA,
            pltpu.SemaphoreType.DMA,
        ],
        compiler_params=pltpu.CompilerParams(use_tc_tiling_on_sc=False),
    )
    def k(x_hbm, lens_hbm, table_hbm, out_hbm, xv, lens_vm, rows, outb,
          sem0, sem1, sem2, sem3, sem4, sem5, sem6, sem7):
        wid = lax.axis_index("s") * NC + lax.axis_index("c")
        base = wid * BPW
        izero = jnp.zeros((16,), jnp.int32)
        zero = jnp.zeros((16,), jnp.float32)
        sems = (sem0, sem1, sem2, sem3, sem4, sem5, sem6, sem7)

        def zpad(b, _):
            xv[b, pl.ds(192, 16)] = izero
            return 0
        lax.fori_loop(0, CB, zpad, 0)

        def fire(b, par):
            ln = lens_vm[b, pl.ds(0, 16)][0]
            nch = (ln + (_GC - 1)) // _GC

            def fbody(c, _):
                pltpu.async_copy(
                    table_hbm.at[xv.at[b, pl.ds(c * _GC, _GC)]],
                    rows.at[par, pl.ds(c * _GC, _GC), :],
                    sems[par],
                )
                return 0
            lax.fori_loop(0, nch, fbody, 0)

        def consume(b, par):
            ln = lens_vm[b, pl.ds(0, 16)][0]
            nch = (ln + (_GC - 1)) // _GC

            def drain(c, _):
                pltpu.make_async_copy(
                    table_hbm.at[pl.ds(0, _GC), :],
                    rows.at[par, pl.ds(0, _GC), :],
                    sems[par],
                ).wait()
                return 0
            lax.fori_loop(0, nch, drain, 0)

            nfull = ln // 16
            rem = ln - nfull * 16

            def acc_full(g, accs):
                a0, a1, b0, b1 = accs
                r0 = g * 16
                for j in range(0, 16, 2):
                    a0 = a0 + rows[par, r0 + j, pl.ds(0, 16)]
                    a1 = a1 + rows[par, r0 + j, pl.ds(16, 16)]
                    b0 = b0 + rows[par, r0 + j + 1, pl.ds(0, 16)]
                    b1 = b1 + rows[par, r0 + j + 1, pl.ds(16, 16)]
                return (a0, a1, b0, b1)

            a0, a1, b0, b1 = lax.fori_loop(0, nfull, acc_full,
                                           (zero, zero, zero, zero))
            a0 = a0 + b0
            a1 = a1 + b1

            def acc_rem(j, accs):
                a0, a1 = accs
                r = nfull * 16 + j
                return (a0 + rows[par, r, pl.ds(0, 16)],
                        a1 + rows[par, r, pl.ds(16, 16)])
            a0, a1 = lax.fori_loop(0, rem, acc_rem, (a0, a1))

            vln = jnp.broadcast_to(ln.astype(jnp.float32), (16,))
            outb[b, pl.ds(0, 16)] = a0 / vln
            outb[b, pl.ds(16, 16)] = a1 / vln

        def do_chunk(cb, _):
            gb0 = base + cb * CB
            pltpu.sync_copy(x_hbm.at[pl.ds(gb0, CB), :], xv.at[:, pl.ds(0, L)])
            pltpu.sync_copy(lens_hbm.at[pl.ds(gb0, CB), :], lens_vm)

            for p in range(6):
                fire(jnp.int32(p), p)

            def do_oct(q, _):
                i0 = q * 8
                for j in range(8):
                    b = i0 + j
                    b6 = b + 6
                    pl.when(b6 < CB)(lambda: fire(b6, (j + 6) % 8))
                    consume(b, j)
                return 0

            lax.fori_loop(0, CB // 8, do_oct, 0)

            pltpu.sync_copy(outb, out_hbm.at[pl.ds(gb0, CB), :])
            return 0

        lax.fori_loop(0, NCB, do_chunk, 0)

    return k


def kernel(x, x_lens, embed_table):
    B, L = x.shape
    V, D = embed_table.shape
    k = _build(B, L, D)
    lens_splat = jnp.broadcast_to(x_lens.astype(jnp.int32)[:, None], (B, 16))
    t4 = lax.optimization_barrier(jnp.reshape(embed_table, (V * D // 128, 128)))
    return k(x.astype(jnp.int32), lens_splat, jnp.reshape(t4, (V, D)))
